# baseline (speedup 1.0000x reference)
import jax
import jax.numpy as jnp
from jax.experimental import pallas as pl

N = 4096
F = 64
H = 64
BM1 = 256
BM2 = 128


def _y_kernel(xn_ref, sg_ref, st_ref, og1_ref, ot1_ref, og2_ref, ot2_ref,
              w_ref, y_ref):
    x = jnp.concatenate(
        [xn_ref[...], sg_ref[...], st_ref[...], og1_ref[...],
         ot1_ref[...], og2_ref[...], ot2_ref[...]], axis=1)
    y_ref[...] = jnp.dot(x, w_ref[...], preferred_element_type=jnp.float32)


def _p_kernel(xp_ref, ap_ref, w13_ref, b13_ref, w210_ref, z11_ref):
    y = jnp.dot(xp_ref[...], w13_ref[...], preferred_element_type=jnp.float32)
    t = jnp.dot(ap_ref[...], y, preferred_element_type=jnp.float32)
    t = jax.nn.relu(t + b13_ref[...])
    s = jnp.sum(t, axis=0)
    smat = jnp.broadcast_to(s[:, None], (H, H))
    z11s = jnp.dot(smat, w210_ref[...], preferred_element_type=jnp.float32)
    z11 = jnp.broadcast_to(z11s[:, None, :], (H, N // H, H))
    z11_ref[...] = z11.reshape(N, H)


def _l1_kernel(a_ref, y_ref, z11_ref, b1_ref, w2_ref, z_ref):
    x1 = jnp.dot(a_ref[...], y_ref[...], preferred_element_type=jnp.float32)
    x1 = jax.nn.relu(x1 + b1_ref[...])
    x1aug = jnp.concatenate([x1, z11_ref[...]], axis=1)
    z_ref[...] = jnp.dot(x1aug, w2_ref[...],
                         preferred_element_type=jnp.float32)


def _l2_kernel(an_ref, ats_ref, acs_ref, as_ref, z_ref, b2_ref, f_ref):
    i = pl.program_id(0)
    pn = jnp.dot(an_ref[...], z_ref[:, 0:512],
                 preferred_element_type=jnp.float32)
    pts = jnp.dot(ats_ref[...], z_ref[:, 512:576],
                  preferred_element_type=jnp.float32)
    pcs = jnp.dot(acs_ref[...], z_ref[:, 576:640],
                  preferred_element_type=jnp.float32)
    ps = jnp.dot(as_ref[...], z_ref[:, 640:704],
                 preferred_element_type=jnp.float32)
    p = jnp.concatenate([pn[:, 0:64], pts, pcs, ps, pn[:, 64:512]], axis=1)
    r = jax.nn.relu(p + b2_ref[...])
    part = jnp.sum(r, axis=0, keepdims=True)

    @pl.when(i == 0)
    def _():
        f_ref[...] = part

    @pl.when(i > 0)
    def _():
        f_ref[...] += part


def kernel(x_n, A_n, A_s, A_n_ts, A_n_cs, mask, x_p, A_p, self_g, self_t,
           other_g1, other_t1, other_g2, other_t2,
           W1, b1, W2, b2, Wd1, bd1, Wd2, bd2, Wo, bo):
    f32 = jnp.float32
    xn, sg, st = x_n[0], self_g[0], self_t[0]
    og1, ot1, og2, ot2 = other_g1[0], other_t1[0], other_g2[0], other_t2[0]
    an, ats, acs, asd = A_n[0], A_n_ts[0], A_n_cs[0], A_s[0]
    xp, ap = x_p[0], A_p[0]

    w1cat = jnp.zeros((448, 576), f32)
    for g, (t, wi) in enumerate(
            [(0, 0), (0, 1), (0, 2), (1, 4), (2, 5), (3, 6), (4, 7),
             (5, 8), (6, 9)]):
        w1cat = w1cat.at[64 * t:64 * (t + 1), 64 * g:64 * (g + 1)].set(W1[wi])
    b1cat = b1[jnp.array([0, 1, 2, 4, 5, 6, 7, 8, 9])].reshape(1, 576)

    w2cat = jnp.zeros((640, 704), f32)
    for h, g, wi in [(0, 0, 0), (1, 8, 1), (1, 9, 2), (2, 10, 3), (3, 1, 4),
                     (4, 2, 5), (5, 3, 6), (6, 4, 7), (7, 5, 8), (8, 6, 9)]:
        w2cat = w2cat.at[64 * h:64 * (h + 1), 64 * g:64 * (g + 1)].set(W2[wi])
    w2cat = w2cat.at[576:640, 448:512].set(jnp.eye(64, dtype=f32))
    b2cat = b2.reshape(1, 704)

    y = pl.pallas_call(
        _y_kernel,
        grid=(N // BM1,),
        in_specs=[pl.BlockSpec((BM1, F), lambda i: (i, 0))] * 7
                 + [pl.BlockSpec((448, 576), lambda i: (0, 0))],
        out_specs=pl.BlockSpec((BM1, 576), lambda i: (i, 0)),
        out_shape=jax.ShapeDtypeStruct((N, 576), f32),
    )(xn, sg, st, og1, ot1, og2, ot2, w1cat)

    z11 = pl.pallas_call(
        _p_kernel,
        out_shape=jax.ShapeDtypeStruct((N, H), f32),
    )(xp, ap, W1[3], b1[3].reshape(1, H), W2[10])

    z = pl.pallas_call(
        _l1_kernel,
        grid=(N // BM1,),
        in_specs=[
            pl.BlockSpec((BM1, N), lambda i: (i, 0)),
            pl.BlockSpec((N, 576), lambda i: (0, 0)),
            pl.BlockSpec((BM1, H), lambda i: (i, 0)),
            pl.BlockSpec((1, 576), lambda i: (0, 0)),
            pl.BlockSpec((640, 704), lambda i: (0, 0)),
        ],
        out_specs=pl.BlockSpec((BM1, 704), lambda i: (i, 0)),
        out_shape=jax.ShapeDtypeStruct((N, 704), f32),
    )(an, y, z11, b1cat, w2cat)

    feats = pl.pallas_call(
        _l2_kernel,
        grid=(N // BM2,),
        in_specs=[pl.BlockSpec((BM2, N), lambda i: (i, 0))] * 4
                 + [pl.BlockSpec((N, 704), lambda i: (0, 0)),
                    pl.BlockSpec((1, 704), lambda i: (0, 0))],
        out_specs=pl.BlockSpec((1, 704), lambda i: (0, 0)),
        out_shape=jax.ShapeDtypeStruct((1, 704), f32),
    )(an, ats, acs, asd, z, b2cat)

    q = jax.nn.relu(jnp.matmul(feats, Wd1) + bd1)
    q = jax.nn.relu(jnp.matmul(q, Wd2) + bd2)
    return jnp.matmul(q, Wo) + bo


# speedup vs baseline: 1.0982x; 1.0982x over previous
import jax
import jax.numpy as jnp
from jax.experimental import pallas as pl

N = 4096
F = 64
H = 64
BM1 = 256
BM2 = 128


def _y_kernel(xn_ref, sg_ref, st_ref, og1_ref, ot1_ref, og2_ref, ot2_ref,
              w_ref, y_ref):
    x = jnp.concatenate(
        [xn_ref[...], sg_ref[...], st_ref[...], og1_ref[...],
         ot1_ref[...], og2_ref[...], ot2_ref[...]], axis=1)
    y = jnp.dot(x.astype(jnp.bfloat16), w_ref[...],
                preferred_element_type=jnp.float32)
    y_ref[...] = y.astype(jnp.bfloat16)


def _p_kernel(xp_ref, ap_ref, w13_ref, b13_ref, w210_ref, z11_ref):
    y = jnp.dot(xp_ref[...], w13_ref[...], preferred_element_type=jnp.float32)
    t = jnp.dot(ap_ref[...], y, preferred_element_type=jnp.float32)
    t = jax.nn.relu(t + b13_ref[...])
    s = jnp.sum(t, axis=0)
    smat = jnp.broadcast_to(s[:, None], (H, H))
    z11s = jnp.dot(smat, w210_ref[...], preferred_element_type=jnp.float32)
    z11 = jnp.broadcast_to(z11s[:, None, :], (H, N // H, H))
    z11_ref[...] = z11.reshape(N, H)


def _l1_kernel(a_ref, y_ref, z11_ref, b1_ref, w2_ref, z_ref):
    x1 = jnp.dot(a_ref[...].astype(jnp.bfloat16), y_ref[...],
                 preferred_element_type=jnp.float32)
    x1 = jax.nn.relu(x1 + b1_ref[...])
    x1aug = jnp.concatenate([x1, z11_ref[...]], axis=1)
    z = jnp.dot(x1aug.astype(jnp.bfloat16), w2_ref[...],
                preferred_element_type=jnp.float32)
    z_ref[...] = z.astype(jnp.bfloat16)


def _l2_kernel(an_ref, ats_ref, acs_ref, as_ref, z_ref, b2_ref, f_ref):
    i = pl.program_id(0)
    b16 = jnp.bfloat16
    pn = jnp.dot(an_ref[...].astype(b16), z_ref[:, 0:512],
                 preferred_element_type=jnp.float32)
    pts = jnp.dot(ats_ref[...].astype(b16), z_ref[:, 512:576],
                  preferred_element_type=jnp.float32)
    pcs = jnp.dot(acs_ref[...].astype(b16), z_ref[:, 576:640],
                  preferred_element_type=jnp.float32)
    ps = jnp.dot(as_ref[...].astype(b16), z_ref[:, 640:704],
                 preferred_element_type=jnp.float32)
    p = jnp.concatenate([pn[:, 0:64], pts, pcs, ps, pn[:, 64:512]], axis=1)
    r = jax.nn.relu(p + b2_ref[...])
    part = jnp.sum(r, axis=0, keepdims=True)

    @pl.when(i == 0)
    def _():
        f_ref[...] = part

    @pl.when(i > 0)
    def _():
        f_ref[...] += part


def kernel(x_n, A_n, A_s, A_n_ts, A_n_cs, mask, x_p, A_p, self_g, self_t,
           other_g1, other_t1, other_g2, other_t2,
           W1, b1, W2, b2, Wd1, bd1, Wd2, bd2, Wo, bo):
    f32 = jnp.float32
    xn, sg, st = x_n[0], self_g[0], self_t[0]
    og1, ot1, og2, ot2 = other_g1[0], other_t1[0], other_g2[0], other_t2[0]
    an, ats, acs, asd = A_n[0], A_n_ts[0], A_n_cs[0], A_s[0]
    xp, ap = x_p[0], A_p[0]

    w1cat = jnp.zeros((448, 576), f32)
    for g, (t, wi) in enumerate(
            [(0, 0), (0, 1), (0, 2), (1, 4), (2, 5), (3, 6), (4, 7),
             (5, 8), (6, 9)]):
        w1cat = w1cat.at[64 * t:64 * (t + 1), 64 * g:64 * (g + 1)].set(W1[wi])
    b1cat = b1[jnp.array([0, 1, 2, 4, 5, 6, 7, 8, 9])].reshape(1, 576)

    w2cat = jnp.zeros((640, 704), f32)
    for h, g, wi in [(0, 0, 0), (1, 8, 1), (1, 9, 2), (2, 10, 3), (3, 1, 4),
                     (4, 2, 5), (5, 3, 6), (6, 4, 7), (7, 5, 8), (8, 6, 9)]:
        w2cat = w2cat.at[64 * h:64 * (h + 1), 64 * g:64 * (g + 1)].set(W2[wi])
    w2cat = w2cat.at[576:640, 448:512].set(jnp.eye(64, dtype=f32))
    b2cat = b2.reshape(1, 704)

    y = pl.pallas_call(
        _y_kernel,
        grid=(N // BM1,),
        in_specs=[pl.BlockSpec((BM1, F), lambda i: (i, 0))] * 7
                 + [pl.BlockSpec((448, 576), lambda i: (0, 0))],
        out_specs=pl.BlockSpec((BM1, 576), lambda i: (i, 0)),
        out_shape=jax.ShapeDtypeStruct((N, 576), jnp.bfloat16),
    )(xn, sg, st, og1, ot1, og2, ot2, w1cat.astype(jnp.bfloat16))

    z11 = pl.pallas_call(
        _p_kernel,
        out_shape=jax.ShapeDtypeStruct((N, H), f32),
    )(xp, ap, W1[3], b1[3].reshape(1, H), W2[10])

    z = pl.pallas_call(
        _l1_kernel,
        grid=(N // BM1,),
        in_specs=[
            pl.BlockSpec((BM1, N), lambda i: (i, 0)),
            pl.BlockSpec((N, 576), lambda i: (0, 0)),
            pl.BlockSpec((BM1, H), lambda i: (i, 0)),
            pl.BlockSpec((1, 576), lambda i: (0, 0)),
            pl.BlockSpec((640, 704), lambda i: (0, 0)),
        ],
        out_specs=pl.BlockSpec((BM1, 704), lambda i: (i, 0)),
        out_shape=jax.ShapeDtypeStruct((N, 704), jnp.bfloat16),
    )(an, y, z11, b1cat, w2cat.astype(jnp.bfloat16))

    feats = pl.pallas_call(
        _l2_kernel,
        grid=(N // BM2,),
        in_specs=[pl.BlockSpec((BM2, N), lambda i: (i, 0))] * 4
                 + [pl.BlockSpec((N, 704), lambda i: (0, 0)),
                    pl.BlockSpec((1, 704), lambda i: (0, 0))],
        out_specs=pl.BlockSpec((1, 704), lambda i: (0, 0)),
        out_shape=jax.ShapeDtypeStruct((1, 704), f32),
    )(an, ats, acs, asd, z, b2cat)

    q = jax.nn.relu(jnp.matmul(feats, Wd1) + bd1)
    q = jax.nn.relu(jnp.matmul(q, Wd2) + bd2)
    return jnp.matmul(q, Wo) + bo


# speedup vs baseline: 1.1259x; 1.0252x over previous
import jax
import jax.numpy as jnp
from jax.experimental import pallas as pl

N = 4096
F = 64
H = 64
BM1 = 512
BM2 = 256


def _y_kernel(xn_ref, sg_ref, st_ref, og1_ref, ot1_ref, og2_ref, ot2_ref,
              w_ref, y_ref):
    x = jnp.concatenate(
        [xn_ref[...], sg_ref[...], st_ref[...], og1_ref[...],
         ot1_ref[...], og2_ref[...], ot2_ref[...]], axis=1)
    y = jnp.dot(x.astype(jnp.bfloat16), w_ref[...],
                preferred_element_type=jnp.float32)
    y_ref[...] = y.astype(jnp.bfloat16)


def _p_kernel(xp_ref, ap_ref, w13_ref, b13_ref, w210_ref, z11_ref):
    y = jnp.dot(xp_ref[...], w13_ref[...], preferred_element_type=jnp.float32)
    t = jnp.dot(ap_ref[...], y, preferred_element_type=jnp.float32)
    t = jax.nn.relu(t + b13_ref[...])
    s = jnp.sum(t, axis=0)
    smat = jnp.broadcast_to(s[:, None], (H, H))
    z11s = jnp.dot(smat, w210_ref[...], preferred_element_type=jnp.float32)
    z11 = jnp.broadcast_to(z11s[:, None, :], (H, N // H, H))
    z11_ref[...] = z11.reshape(N, H)


def _l1_kernel(a_ref, y_ref, z11_ref, b1_ref, w2_ref, z_ref):
    x1 = jnp.dot(a_ref[...].astype(jnp.bfloat16), y_ref[...],
                 preferred_element_type=jnp.float32)
    x1 = jax.nn.relu(x1 + b1_ref[...])
    x1aug = jnp.concatenate([x1, z11_ref[...]], axis=1)
    z = jnp.dot(x1aug.astype(jnp.bfloat16), w2_ref[...],
                preferred_element_type=jnp.float32)
    z_ref[...] = z.astype(jnp.bfloat16)


def _l2_kernel(an_ref, ats_ref, acs_ref, as_ref, z_ref, b2_ref, f_ref):
    i = pl.program_id(0)
    b16 = jnp.bfloat16
    pn = jnp.dot(an_ref[...].astype(b16), z_ref[:, 0:512],
                 preferred_element_type=jnp.float32)
    pts = jnp.dot(ats_ref[...].astype(b16), z_ref[:, 512:576],
                  preferred_element_type=jnp.float32)
    pcs = jnp.dot(acs_ref[...].astype(b16), z_ref[:, 576:640],
                  preferred_element_type=jnp.float32)
    ps = jnp.dot(as_ref[...].astype(b16), z_ref[:, 640:704],
                 preferred_element_type=jnp.float32)
    p = jnp.concatenate([pn[:, 0:64], pts, pcs, ps, pn[:, 64:512]], axis=1)
    r = jax.nn.relu(p + b2_ref[...])
    part = jnp.sum(r, axis=0, keepdims=True)

    @pl.when(i == 0)
    def _():
        f_ref[...] = part

    @pl.when(i > 0)
    def _():
        f_ref[...] += part


def kernel(x_n, A_n, A_s, A_n_ts, A_n_cs, mask, x_p, A_p, self_g, self_t,
           other_g1, other_t1, other_g2, other_t2,
           W1, b1, W2, b2, Wd1, bd1, Wd2, bd2, Wo, bo):
    f32 = jnp.float32
    xn, sg, st = x_n[0], self_g[0], self_t[0]
    og1, ot1, og2, ot2 = other_g1[0], other_t1[0], other_g2[0], other_t2[0]
    an, ats, acs, asd = A_n[0], A_n_ts[0], A_n_cs[0], A_s[0]
    xp, ap = x_p[0], A_p[0]

    w1cat = jnp.zeros((448, 576), f32)
    for g, (t, wi) in enumerate(
            [(0, 0), (0, 1), (0, 2), (1, 4), (2, 5), (3, 6), (4, 7),
             (5, 8), (6, 9)]):
        w1cat = w1cat.at[64 * t:64 * (t + 1), 64 * g:64 * (g + 1)].set(W1[wi])
    b1cat = b1[jnp.array([0, 1, 2, 4, 5, 6, 7, 8, 9])].reshape(1, 576)

    w2cat = jnp.zeros((640, 704), f32)
    for h, g, wi in [(0, 0, 0), (1, 8, 1), (1, 9, 2), (2, 10, 3), (3, 1, 4),
                     (4, 2, 5), (5, 3, 6), (6, 4, 7), (7, 5, 8), (8, 6, 9)]:
        w2cat = w2cat.at[64 * h:64 * (h + 1), 64 * g:64 * (g + 1)].set(W2[wi])
    w2cat = w2cat.at[576:640, 448:512].set(jnp.eye(64, dtype=f32))
    b2cat = b2.reshape(1, 704)

    y = pl.pallas_call(
        _y_kernel,
        grid=(N // BM1,),
        in_specs=[pl.BlockSpec((BM1, F), lambda i: (i, 0))] * 7
                 + [pl.BlockSpec((448, 576), lambda i: (0, 0))],
        out_specs=pl.BlockSpec((BM1, 576), lambda i: (i, 0)),
        out_shape=jax.ShapeDtypeStruct((N, 576), jnp.bfloat16),
    )(xn, sg, st, og1, ot1, og2, ot2, w1cat.astype(jnp.bfloat16))

    z11 = pl.pallas_call(
        _p_kernel,
        out_shape=jax.ShapeDtypeStruct((N, H), f32),
    )(xp, ap, W1[3], b1[3].reshape(1, H), W2[10])

    z = pl.pallas_call(
        _l1_kernel,
        grid=(N // BM1,),
        in_specs=[
            pl.BlockSpec((BM1, N), lambda i: (i, 0)),
            pl.BlockSpec((N, 576), lambda i: (0, 0)),
            pl.BlockSpec((BM1, H), lambda i: (i, 0)),
            pl.BlockSpec((1, 576), lambda i: (0, 0)),
            pl.BlockSpec((640, 704), lambda i: (0, 0)),
        ],
        out_specs=pl.BlockSpec((BM1, 704), lambda i: (i, 0)),
        out_shape=jax.ShapeDtypeStruct((N, 704), jnp.bfloat16),
    )(an, y, z11, b1cat, w2cat.astype(jnp.bfloat16))

    feats = pl.pallas_call(
        _l2_kernel,
        grid=(N // BM2,),
        in_specs=[pl.BlockSpec((BM2, N), lambda i: (i, 0))] * 4
                 + [pl.BlockSpec((N, 704), lambda i: (0, 0)),
                    pl.BlockSpec((1, 704), lambda i: (0, 0))],
        out_specs=pl.BlockSpec((1, 704), lambda i: (0, 0)),
        out_shape=jax.ShapeDtypeStruct((1, 704), f32),
    )(an, ats, acs, asd, z, b2cat)

    q = jax.nn.relu(jnp.matmul(feats, Wd1) + bd1)
    q = jax.nn.relu(jnp.matmul(q, Wd2) + bd2)
    return jnp.matmul(q, Wo) + bo
